# Initial kernel scaffold; baseline (speedup 1.0000x reference)
import jax
import jax.numpy as jnp
from jax.experimental import pallas as pl


def kernel(champion_ids, role, embed, W1, b1, g1, be1, W2, b2, g2, be2, W3, b3, g3, be3, W4, b4):
    raise NotImplementedError("write your pallas kernel here")


# speedup vs baseline: 9.9866x; 9.9866x over previous
import functools

import jax
import jax.numpy as jnp
from jax.experimental import pallas as pl
from jax.experimental.pallas import tpu as pltpu

_R = 2048


def _ln_relu(h, g, be):
    m = jnp.mean(h, axis=-1, keepdims=True)
    c = h - m
    v = jnp.mean(c * c, axis=-1, keepdims=True)
    return jnp.maximum(c * jax.lax.rsqrt(v + 1e-5) * g + be, 0.0)


def _mlp_body(ids_ref, role_ref, table_ref, w1a_ref, w1b_ref, b1_ref, g1_ref,
              be1_ref, w2_ref, b2_ref, g2_ref, be2_ref, w3_ref, b3_ref,
              g3_ref, be3_ref, w4t_ref, b4_ref, out_ref):
    ids = ids_ref[...]
    table = table_ref[...]
    embs = []
    for j in range(11):
        oh = (ids[:, j:j + 1] == jax.lax.broadcasted_iota(
            jnp.int32, (1, 166), 1)).astype(jnp.float32)
        embs.append(jnp.dot(oh, table, preferred_element_type=jnp.float32))
    emb = jnp.concatenate(embs, axis=1)
    h = jnp.dot(emb, w1a_ref[...], preferred_element_type=jnp.float32)
    h = h + jnp.dot(role_ref[...], w1b_ref[...],
                    preferred_element_type=jnp.float32)
    h = _ln_relu(h + b1_ref[...], g1_ref[...], be1_ref[...])
    h = jnp.dot(h, w2_ref[...], preferred_element_type=jnp.float32)
    h = _ln_relu(h + b2_ref[...], g2_ref[...], be2_ref[...])
    h = jnp.dot(h, w3_ref[...], preferred_element_type=jnp.float32)
    h = _ln_relu(h + b3_ref[...], g3_ref[...], be3_ref[...])
    o = jnp.sum(h * w4t_ref[...], axis=1, keepdims=True) + b4_ref[...]
    out_ref[...] = jax.nn.sigmoid(o)


def kernel(champion_ids, role, embed, W1, b1, g1, be1, W2, b2, g2, be2, W3,
           b3, g3, be3, W4, b4):
    B = champion_ids.shape[0]
    table = embed.at[0].set(0.0)
    w1a, w1b = W1[:352], W1[352:]
    row = lambda v: v.reshape(1, -1)
    grid = B // _R
    tile = lambda i: (i, 0)
    rep = lambda i: (0, 0)
    out = pl.pallas_call(
        _mlp_body,
        grid=(grid,),
        in_specs=[
            pl.BlockSpec((_R, 11), tile),
            pl.BlockSpec((_R, 5), tile),
        ] + [pl.BlockSpec(w.shape, rep) for w in (
            table, w1a, w1b, row(b1), row(g1), row(be1), W2, row(b2),
            row(g2), row(be2), W3, row(b3), row(g3), row(be3),
            W4.reshape(1, -1), row(b4))],
        out_specs=pl.BlockSpec((_R, 1), tile),
        out_shape=jax.ShapeDtypeStruct((B, 1), jnp.float32),
    )(champion_ids, role, table, w1a, w1b, row(b1), row(g1), row(be1), W2,
      row(b2), row(g2), row(be2), W3, row(b3), row(g3), row(be3),
      W4.reshape(1, -1), row(b4))
    return out[:, 0]
